# baseline (speedup 1.0000x reference)
import functools

import jax
import jax.numpy as jnp
from jax import lax
from jax.experimental import pallas as pl
from jax.experimental.pallas import tpu as pltpu
from jax.experimental.pallas import tpu_sc as plsc

VOCAB = 100000
D = 128
B_ROWS = 4096 * 200


_PROJ_BLK = 20000


def _proj_body(t_ref, w_ref, b_ref, o_ref):
    o_ref[...] = lax.dot_general(
        t_ref[...], w_ref[...],
        (((1,), (1,)), ((), ())),
        preferred_element_type=jnp.float32,
    ) + b_ref[...]


def _project_table(table, W, b):
    grid = (VOCAB // _PROJ_BLK,)
    return pl.pallas_call(
        _proj_body,
        grid=grid,
        in_specs=[
            pl.BlockSpec((_PROJ_BLK, D), lambda i: (i, 0)),
            pl.BlockSpec((D, D), lambda i: (0, 0)),
            pl.BlockSpec((1, D), lambda i: (0, 0)),
        ],
        out_specs=pl.BlockSpec((_PROJ_BLK, D), lambda i: (i, 0)),
        out_shape=jax.ShapeDtypeStruct((VOCAB, D), jnp.float32),
    )(table, W, b.reshape(1, D))


_NW = 32
_CHUNK = 128
_PER_W = B_ROWS // _NW
_NCHUNK = _PER_W // _CHUNK


_NB = 2
_PF = 1
_GPS = 2
_SUPER = _CHUNK * _GPS
_NSUPER = _PER_W // _SUPER


def _gather_body(ptab_hbm, xw_hbm, out_hbm, idx_v,
                 rows0, rows1,
                 g0, g1, s0, s1):
    wid = lax.axis_index("s") * 2 + lax.axis_index("c")
    pltpu.sync_copy(xw_hbm.at[wid], idx_v)
    base_row = wid * _PER_W
    rows = (rows0, rows1)
    gsem = (g0, g1)
    ssem = (s0, s1)

    def gather_start(j, b):
        for g in range(_GPS):
            pltpu.async_copy(ptab_hbm.at[idx_v.at[j * _GPS + g]],
                             rows[b].at[pl.ds(g * _CHUNK, _CHUNK)], gsem[b])

    def gather_drain(b):
        pltpu.make_async_copy(ptab_hbm.at[pl.ds(0, _SUPER)], rows[b], gsem[b]).wait()

    def store_start(j, b):
        pltpu.async_copy(rows[b], out_hbm.at[pl.ds(base_row + j * _SUPER, _SUPER)],
                         ssem[b])

    def store_drain(b):
        pltpu.make_async_copy(rows[b], out_hbm.at[pl.ds(0, _SUPER)], ssem[b]).wait()

    for p in range(_PF):
        gather_start(p, p)

    def body(o, carry):
        for blk in range(_NB):
            j = o * _NB + blk
            s = blk
            u = (blk + _PF) % _NB

            @pl.when(jnp.logical_and(j + _PF < _NSUPER, j >= 1))
            def _():
                store_drain(u)

            @pl.when(j + _PF < _NSUPER)
            def _():
                gather_start(j + _PF, u)

            gather_drain(s)
            store_start(j, s)
        return carry

    lax.fori_loop(0, _NSUPER // _NB, body, 0, unroll=False)
    for j in range(_NSUPER - _PF - 1, _NSUPER):
        store_drain(j % _NB)


def _sc_gather(ptab, x_flat2d):
    mesh = plsc.VectorSubcoreMesh(core_axis_name="c", subcore_axis_name="s")
    kern = functools.partial(
        pl.kernel,
        mesh=mesh,
        out_type=jax.ShapeDtypeStruct((B_ROWS, D), jnp.float32),
        scratch_types=(
            [pltpu.VMEM((_NCHUNK, _CHUNK), jnp.int32)]
            + [pltpu.VMEM((_SUPER, D), jnp.float32)] * _NB
            + [pltpu.SemaphoreType.DMA] * (2 * _NB)
        ),
    )(_gather_body)
    return kern(ptab, x_flat2d)


def _dummy_body(a_ref, o_ref):
    acc = jnp.zeros((512, 512), jnp.float32)
    for _ in range(8):
        acc = acc + lax.dot_general(a_ref[...], a_ref[...],
                                    (((1,), (0,)), ((), ())),
                                    preferred_element_type=jnp.float32)
    o_ref[...] = acc


def _dummy_tc(table):
    a = table[:512, :128]
    a = jnp.concatenate([a, a, a, a], axis=1)
    return pl.pallas_call(
        _dummy_body,
        grid=(64,),
        in_specs=[pl.BlockSpec((512, 512), lambda i: (0, 0))],
        out_specs=pl.BlockSpec((512, 512), lambda i: (0, 0)),
        out_shape=jax.ShapeDtypeStruct((512, 512), jnp.float32),
    )(a)


def kernel(x, table, W, b):
    ptab = _project_table(table, W, b)
    x_flat2d = x.reshape(_NW, _NCHUNK, _CHUNK).astype(jnp.int32)
    out = _sc_gather(ptab, x_flat2d)
    dummy = _dummy_tc(table)
    out = out + 0.0 * dummy[0, 0]
    return out.reshape(x.shape[0], x.shape[1], D)


# speedup vs baseline: 1.7244x; 1.7244x over previous
import functools

import jax
import jax.numpy as jnp
from jax import lax
from jax.experimental import pallas as pl
from jax.experimental.pallas import tpu as pltpu
from jax.experimental.pallas import tpu_sc as plsc

VOCAB = 100000
D = 128
B_ROWS = 4096 * 200


_PROJ_BLK = 20000


def _proj_body(t_ref, w_ref, b_ref, o_ref):
    o_ref[...] = lax.dot_general(
        t_ref[...], w_ref[...],
        (((1,), (1,)), ((), ())),
        preferred_element_type=jnp.float32,
    ) + b_ref[...]


def _project_table(table, W, b):
    grid = (VOCAB // _PROJ_BLK,)
    return pl.pallas_call(
        _proj_body,
        grid=grid,
        in_specs=[
            pl.BlockSpec((_PROJ_BLK, D), lambda i: (i, 0)),
            pl.BlockSpec((D, D), lambda i: (0, 0)),
            pl.BlockSpec((1, D), lambda i: (0, 0)),
        ],
        out_specs=pl.BlockSpec((_PROJ_BLK, D), lambda i: (i, 0)),
        out_shape=jax.ShapeDtypeStruct((VOCAB, D), jnp.float32),
    )(table, W, b.reshape(1, D))


_NW = 32
_CHUNK = 128
_PER_W = B_ROWS // _NW
_NCHUNK = _PER_W // _CHUNK


_NB = 2
_PF = 1
_GPS = 2
_SUPER = _CHUNK * _GPS
_NSUPER = _PER_W // _SUPER


def _gather_body(ptab_hbm, xw_hbm, out_hbm, idx_v,
                 rows0, rows1,
                 g0, g1, s0, s1):
    wid = lax.axis_index("s") * 2 + lax.axis_index("c")
    pltpu.sync_copy(xw_hbm.at[wid], idx_v)
    base_row = wid * _PER_W
    rows = (rows0, rows1)
    gsem = (g0, g1)
    ssem = (s0, s1)

    def gather_start(j, b):
        for g in range(_GPS):
            pltpu.async_copy(ptab_hbm.at[idx_v.at[j * _GPS + g]],
                             rows[b].at[pl.ds(g * _CHUNK, _CHUNK)], gsem[b])

    def gather_drain(b):
        pltpu.make_async_copy(ptab_hbm.at[pl.ds(0, _SUPER)], rows[b], gsem[b]).wait()

    def store_start(j, b):
        pltpu.async_copy(rows[b], out_hbm.at[pl.ds(base_row + j * _SUPER, _SUPER)],
                         ssem[b])

    def store_drain(b):
        pltpu.make_async_copy(rows[b], out_hbm.at[pl.ds(0, _SUPER)], ssem[b]).wait()

    for p in range(_PF):
        gather_start(p, p)

    def body(o, carry):
        for blk in range(_NB):
            j = o * _NB + blk
            s = blk
            u = (blk + _PF) % _NB

            @pl.when(jnp.logical_and(j + _PF < _NSUPER, j >= 1))
            def _():
                store_drain(u)

            @pl.when(j + _PF < _NSUPER)
            def _():
                gather_start(j + _PF, u)

            gather_drain(s)
            store_start(j, s)
        return carry

    lax.fori_loop(0, _NSUPER // _NB, body, 0, unroll=False)
    for j in range(_NSUPER - _PF - 1, _NSUPER):
        store_drain(j % _NB)


def _sc_gather(ptab, x_flat2d):
    mesh = plsc.VectorSubcoreMesh(core_axis_name="c", subcore_axis_name="s")
    kern = functools.partial(
        pl.kernel,
        mesh=mesh,
        out_type=jax.ShapeDtypeStruct((B_ROWS, D), jnp.float32),
        scratch_types=(
            [pltpu.VMEM((_NCHUNK, _CHUNK), jnp.int32)]
            + [pltpu.VMEM((_SUPER, D), jnp.float32)] * _NB
            + [pltpu.SemaphoreType.DMA] * (2 * _NB)
        ),
    )(_gather_body)
    return kern(ptab, x_flat2d)


def kernel(x, table, W, b):
    ptab = _project_table(table, W, b)
    x_flat2d = x.reshape(_NW, _NCHUNK, _CHUNK).astype(jnp.int32)
    out = _sc_gather(ptab, x_flat2d)
    return out.reshape(x.shape[0], x.shape[1], D)
